# baseline (speedup 1.0000x reference)
import jax
import jax.numpy as jnp
from jax.experimental import pallas as pl
from jax.experimental.pallas import tpu as pltpu

B = 4096
EMB = 128
GD = 300
E = 8
T = 12
AE = E * T
BLK = 512
NB = B // BLK


def _fused(xs_ref, xg_ref, rs_ref, rg_ref,
           sw1_ref, sb1_ref, sgm_ref, sbt_ref, sw2_ref, sb2_ref,
           gw1_ref, gb1_ref, ggm_ref, gbt_ref, gw2_ref, gb2_ref,
           sc_ref, gc_ref, sew_ref, seb_ref, gew_ref, geb_ref,
           r_ref, s_ref, out_ref,
           hs_scr, hg_scr, s1s_scr, s2s_scr, s1g_scr, s2g_scr):
    p = pl.program_id(0)
    k = pl.program_id(1)

    @pl.when(p == 0)
    def _phase0():
        def stats(x_ref, w1_ref, b1_ref, h_scr, s1_scr, s2_scr):
            h = jnp.dot(x_ref[...], w1_ref[...],
                        preferred_element_type=jnp.float32) + b1_ref[...]
            h_scr[pl.ds(k * BLK, BLK), :] = h
            s1 = jnp.sum(h, axis=0, keepdims=True)
            s2 = jnp.sum(h * h, axis=0, keepdims=True)

            @pl.when(k == 0)
            def _init():
                s1_scr[...] = s1
                s2_scr[...] = s2

            @pl.when(k > 0)
            def _accum():
                s1_scr[...] += s1
                s2_scr[...] += s2

        stats(xs_ref, sw1_ref, sb1_ref, hs_scr, s1s_scr, s2s_scr)
        stats(xg_ref, gw1_ref, gb1_ref, hg_scr, s1g_scr, s2g_scr)

    @pl.when(p == 1)
    def _phase1():
        def half(rep, h_scr, s1_scr, s2_scr, gamma, beta, w2, b2, clu,
                 ew, eb):
            mu = s1_scr[...] * (1.0 / B)
            var = s2_scr[...] * (1.0 / B) - mu * mu
            scale = gamma * jax.lax.rsqrt(var + 1e-5)
            shift = beta - mu * scale
            h = h_scr[pl.ds(k * BLK, BLK), :] * scale + shift
            h = jnp.maximum(h, 0.0)
            ge = jnp.dot(h, w2, preferred_element_type=jnp.float32) + b2
            logit = jax.lax.dot_general(
                ge, clu, (((1,), (1,)), ((), ())),
                preferred_element_type=jnp.float32)
            m = jnp.max(logit, axis=-1, keepdims=True)
            ex = jnp.exp(logit - m)
            q = ex / jnp.sum(ex, axis=-1, keepdims=True)
            z = jnp.dot(rep, ew, preferred_element_type=jnp.float32) + eb
            qe = jnp.dot(q, r_ref[...], preferred_element_type=jnp.float32)
            return jnp.dot(qe * z, s_ref[...],
                           preferred_element_type=jnp.float32)

        sca = half(rs_ref[...], hs_scr, s1s_scr, s2s_scr,
                   sgm_ref[...], sbt_ref[...], sw2_ref[...], sb2_ref[...],
                   sc_ref[...], sew_ref[...], seb_ref[...])
        gro = half(rg_ref[...], hg_scr, s1g_scr, s2g_scr,
                   ggm_ref[...], gbt_ref[...], gw2_ref[...], gb2_ref[...],
                   gc_ref[...], gew_ref[...], geb_ref[...])
        out_ref[...] = 0.5 * (sca + gro)


def kernel(sca_rep, gro_rep, sca_gate_input, gro_gate_input,
           sca_g_w1, sca_g_b1, sca_g_gamma, sca_g_beta, sca_g_w2, sca_g_b2,
           gro_g_w1, gro_g_b1, gro_g_gamma, gro_g_beta, gro_g_w2, gro_g_b2,
           sca_cluster, gro_cluster,
           sca_experts_w, sca_experts_b, gro_experts_w, gro_experts_b):
    r_mat = jnp.repeat(jnp.eye(E, dtype=jnp.float32), T, axis=1)
    s_mat = jnp.tile(jnp.eye(T, dtype=jnp.float32), (E, 1))

    row_blk = lambda p, k: (k, 0)
    whole = lambda shape: pl.BlockSpec(shape, lambda p, k: (0,) * len(shape))

    out = pl.pallas_call(
        _fused,
        grid=(2, NB),
        in_specs=[
            pl.BlockSpec((BLK, EMB), row_blk),
            pl.BlockSpec((BLK, EMB), row_blk),
            pl.BlockSpec((BLK, EMB), row_blk),
            pl.BlockSpec((BLK, EMB), row_blk),
            whole((EMB, GD)), whole((GD,)), whole((GD,)), whole((GD,)),
            whole((GD, GD)), whole((GD,)),
            whole((EMB, GD)), whole((GD,)), whole((GD,)), whole((GD,)),
            whole((GD, GD)), whole((GD,)),
            whole((E, GD)), whole((E, GD)),
            whole((EMB, AE)), whole((AE,)), whole((EMB, AE)), whole((AE,)),
            whole((E, AE)), whole((AE, T)),
        ],
        out_specs=pl.BlockSpec((BLK, T), row_blk),
        out_shape=jax.ShapeDtypeStruct((B, T), jnp.float32),
        scratch_shapes=[
            pltpu.VMEM((B, GD), jnp.float32),
            pltpu.VMEM((B, GD), jnp.float32),
            pltpu.VMEM((1, GD), jnp.float32),
            pltpu.VMEM((1, GD), jnp.float32),
            pltpu.VMEM((1, GD), jnp.float32),
            pltpu.VMEM((1, GD), jnp.float32),
        ],
        compiler_params=pltpu.CompilerParams(
            dimension_semantics=("arbitrary", "arbitrary"),
            vmem_limit_bytes=120 * 1024 * 1024),
    )(sca_gate_input, gro_gate_input, sca_rep, gro_rep,
      sca_g_w1, sca_g_b1, sca_g_gamma, sca_g_beta, sca_g_w2, sca_g_b2,
      gro_g_w1, gro_g_b1, gro_g_gamma, gro_g_beta, gro_g_w2, gro_g_b2,
      sca_cluster, gro_cluster,
      sca_experts_w, sca_experts_b, gro_experts_w, gro_experts_b,
      r_mat, s_mat)
    return out


# speedup vs baseline: 1.0206x; 1.0206x over previous
import jax
import jax.numpy as jnp
from jax.experimental import pallas as pl
from jax.experimental.pallas import tpu as pltpu

B = 4096
EMB = 128
GD = 300
E = 8
T = 12
AE = E * T
BLK = 512
NB = B // BLK


def _fused(xs_ref, xg_ref, rs_ref, rg_ref,
           sw1_ref, sb1_ref, sgm_ref, sbt_ref, sw2_ref, sb2_ref,
           gw1_ref, gb1_ref, ggm_ref, gbt_ref, gw2_ref, gb2_ref,
           sc_ref, gc_ref, sew_ref, seb_ref, gew_ref, geb_ref,
           r_ref, s_ref, out_ref,
           hs_scr, hg_scr, s1s_scr, s2s_scr, s1g_scr, s2g_scr):
    p = pl.program_id(0)
    k = pl.program_id(1)

    @pl.when(p == 0)
    def _phase0():
        def stats(x_ref, w1_ref, b1_ref, h_scr, s1_scr, s2_scr):
            h = jnp.dot(x_ref[...], w1_ref[...],
                        preferred_element_type=jnp.float32) + b1_ref[...]
            h_scr[pl.ds(k * BLK, BLK), :] = h
            s1 = jnp.sum(h, axis=0, keepdims=True)
            s2 = jnp.sum(h * h, axis=0, keepdims=True)

            @pl.when(k == 0)
            def _init():
                s1_scr[...] = s1
                s2_scr[...] = s2

            @pl.when(k > 0)
            def _accum():
                s1_scr[...] += s1
                s2_scr[...] += s2

        stats(xs_ref, sw1_ref, sb1_ref, hs_scr, s1s_scr, s2s_scr)
        stats(xg_ref, gw1_ref, gb1_ref, hg_scr, s1g_scr, s2g_scr)

    @pl.when(p == 1)
    def _phase1():
        def half(rep, h_scr, s1_scr, s2_scr, gamma, beta, w2, b2, clu,
                 ew, eb):
            mu = s1_scr[...] * (1.0 / B)
            var = s2_scr[...] * (1.0 / B) - mu * mu
            scale = gamma * jax.lax.rsqrt(var + 1e-5)
            shift = beta - mu * scale
            h = h_scr[pl.ds(k * BLK, BLK), :] * scale + shift
            h = jnp.maximum(h, 0.0)
            ge = jnp.dot(h, w2, preferred_element_type=jnp.float32) + b2
            logit = jax.lax.dot_general(
                ge, clu, (((1,), (1,)), ((), ())),
                preferred_element_type=jnp.float32)
            m = jnp.max(logit, axis=-1, keepdims=True)
            ex = jnp.exp(logit - m)
            q = ex / jnp.sum(ex, axis=-1, keepdims=True)
            z = jnp.dot(rep, ew, preferred_element_type=jnp.float32) + eb
            qe = jnp.dot(q, r_ref[...], preferred_element_type=jnp.float32)
            return jnp.dot(qe * z, s_ref[...],
                           preferred_element_type=jnp.float32)

        sca = half(rs_ref[...], hs_scr, s1s_scr, s2s_scr,
                   sgm_ref[...], sbt_ref[...], sw2_ref[...], sb2_ref[...],
                   sc_ref[...], sew_ref[...], seb_ref[...])
        gro = half(rg_ref[...], hg_scr, s1g_scr, s2g_scr,
                   ggm_ref[...], gbt_ref[...], gw2_ref[...], gb2_ref[...],
                   gc_ref[...], gew_ref[...], geb_ref[...])
        out_ref[...] = 0.5 * (sca + gro)


def kernel(sca_rep, gro_rep, sca_gate_input, gro_gate_input,
           sca_g_w1, sca_g_b1, sca_g_gamma, sca_g_beta, sca_g_w2, sca_g_b2,
           gro_g_w1, gro_g_b1, gro_g_gamma, gro_g_beta, gro_g_w2, gro_g_b2,
           sca_cluster, gro_cluster,
           sca_experts_w, sca_experts_b, gro_experts_w, gro_experts_b):
    r_mat = jnp.repeat(jnp.eye(E, dtype=jnp.float32), T, axis=1)
    s_mat = jnp.tile(jnp.eye(T, dtype=jnp.float32), (E, 1))

    gate_blk = lambda p, k: (jnp.where(p == 0, k, NB - 1), 0)
    rep_blk = lambda p, k: (jnp.where(p == 0, 0, k), 0)
    row_blk = lambda p, k: (k, 0)
    whole = lambda shape: pl.BlockSpec(shape, lambda p, k: (0,) * len(shape))

    out = pl.pallas_call(
        _fused,
        grid=(2, NB),
        in_specs=[
            pl.BlockSpec((BLK, EMB), gate_blk),
            pl.BlockSpec((BLK, EMB), gate_blk),
            pl.BlockSpec((BLK, EMB), rep_blk),
            pl.BlockSpec((BLK, EMB), rep_blk),
            whole((EMB, GD)), whole((GD,)), whole((GD,)), whole((GD,)),
            whole((GD, GD)), whole((GD,)),
            whole((EMB, GD)), whole((GD,)), whole((GD,)), whole((GD,)),
            whole((GD, GD)), whole((GD,)),
            whole((E, GD)), whole((E, GD)),
            whole((EMB, AE)), whole((AE,)), whole((EMB, AE)), whole((AE,)),
            whole((E, AE)), whole((AE, T)),
        ],
        out_specs=pl.BlockSpec((BLK, T), row_blk),
        out_shape=jax.ShapeDtypeStruct((B, T), jnp.float32),
        scratch_shapes=[
            pltpu.VMEM((B, GD), jnp.float32),
            pltpu.VMEM((B, GD), jnp.float32),
            pltpu.VMEM((1, GD), jnp.float32),
            pltpu.VMEM((1, GD), jnp.float32),
            pltpu.VMEM((1, GD), jnp.float32),
            pltpu.VMEM((1, GD), jnp.float32),
        ],
        compiler_params=pltpu.CompilerParams(
            dimension_semantics=("arbitrary", "arbitrary"),
            vmem_limit_bytes=120 * 1024 * 1024),
    )(sca_gate_input, gro_gate_input, sca_rep, gro_rep,
      sca_g_w1, sca_g_b1, sca_g_gamma, sca_g_beta, sca_g_w2, sca_g_b2,
      gro_g_w1, gro_g_b1, gro_g_gamma, gro_g_beta, gro_g_w2, gro_g_b2,
      sca_cluster, gro_cluster,
      sca_experts_w, sca_experts_b, gro_experts_w, gro_experts_b,
      r_mat, s_mat)
    return out


# speedup vs baseline: 1.5161x; 1.4856x over previous
import jax
import jax.numpy as jnp
from jax.experimental import pallas as pl
from jax.experimental.pallas import tpu as pltpu

B = 4096
EMB = 128
GD = 300
E = 8
T = 12
AE = E * T
BLK = 512
NB = B // BLK


def _fused(xs_ref, xg_ref, rs_ref, rg_ref,
           sw1t_ref, sb1_ref, sgm_ref, sbt_ref, sw2_ref, sb2_ref,
           gw1t_ref, gb1_ref, ggm_ref, gbt_ref, gw2_ref, gb2_ref,
           sc_ref, gc_ref, sewt_ref, seb_ref, gewt_ref, geb_ref,
           out_ref,
           hs_scr, hg_scr, s1s_scr, s2s_scr, s1g_scr, s2g_scr):
    p = pl.program_id(0)
    k = pl.program_id(1)

    @pl.when(p == 0)
    def _phase0():
        def stats(x_ref, w1t_ref, b1_ref, h_scr, s1_scr, s2_scr):
            h = jax.lax.dot_general(
                x_ref[...], w1t_ref[...], (((1,), (1,)), ((), ())),
                preferred_element_type=jnp.float32) + b1_ref[...]
            h_scr[pl.ds(k * BLK, BLK), :] = h
            s1 = jnp.sum(h, axis=0, keepdims=True)
            s2 = jnp.sum(h * h, axis=0, keepdims=True)

            @pl.when(k == 0)
            def _init():
                s1_scr[...] = s1
                s2_scr[...] = s2

            @pl.when(k > 0)
            def _accum():
                s1_scr[...] += s1
                s2_scr[...] += s2

        stats(xs_ref, sw1t_ref, sb1_ref, hs_scr, s1s_scr, s2s_scr)
        stats(xg_ref, gw1t_ref, gb1_ref, hg_scr, s1g_scr, s2g_scr)

    @pl.when(p == 1)
    def _phase1():
        ie = jax.lax.broadcasted_iota(jnp.int32, (E, AE), 0)
        ja = jax.lax.broadcasted_iota(jnp.int32, (E, AE), 1)
        r_mat = (ja // T == ie).astype(jnp.float32)
        it = jax.lax.broadcasted_iota(jnp.int32, (T, AE), 0)
        jb = jax.lax.broadcasted_iota(jnp.int32, (T, AE), 1)
        st_mat = (jb % T == it).astype(jnp.float32)

        def half(rep, h_scr, s1_scr, s2_scr, gamma, beta, w2, b2, clu,
                 ewt, eb):
            mu = s1_scr[...] * (1.0 / B)
            var = s2_scr[...] * (1.0 / B) - mu * mu
            scale = gamma * jax.lax.rsqrt(var + 1e-5)
            shift = beta - mu * scale
            h = h_scr[pl.ds(k * BLK, BLK), :] * scale + shift
            h = jnp.maximum(h, 0.0)
            ge = jnp.dot(h, w2, preferred_element_type=jnp.float32) + b2
            logit = jax.lax.dot_general(
                ge, clu, (((1,), (1,)), ((), ())),
                preferred_element_type=jnp.float32)
            m = jnp.max(logit, axis=-1, keepdims=True)
            ex = jnp.exp(logit - m)
            q = ex / jnp.sum(ex, axis=-1, keepdims=True)
            z = jax.lax.dot_general(
                rep, ewt, (((1,), (1,)), ((), ())),
                preferred_element_type=jnp.float32) + eb
            qe = jnp.dot(q, r_mat, preferred_element_type=jnp.float32)
            return jax.lax.dot_general(
                st_mat, qe * z, (((1,), (1,)), ((), ())),
                preferred_element_type=jnp.float32)

        sca = half(rs_ref[...], hs_scr, s1s_scr, s2s_scr,
                   sgm_ref[...], sbt_ref[...], sw2_ref[...], sb2_ref[...],
                   sc_ref[...], sewt_ref[...], seb_ref[...])
        gro = half(rg_ref[...], hg_scr, s1g_scr, s2g_scr,
                   ggm_ref[...], gbt_ref[...], gw2_ref[...], gb2_ref[...],
                   gc_ref[...], gewt_ref[...], geb_ref[...])
        out_ref[...] = 0.5 * (sca + gro)


def kernel(sca_rep, gro_rep, sca_gate_input, gro_gate_input,
           sca_g_w1, sca_g_b1, sca_g_gamma, sca_g_beta, sca_g_w2, sca_g_b2,
           gro_g_w1, gro_g_b1, gro_g_gamma, gro_g_beta, gro_g_w2, gro_g_b2,
           sca_cluster, gro_cluster,
           sca_experts_w, sca_experts_b, gro_experts_w, gro_experts_b):
    gate_blk = lambda p, k: (jnp.where(p == 0, k, NB - 1), 0)
    rep_blk = lambda p, k: (jnp.where(p == 0, 0, k), 0)
    whole = lambda shape: pl.BlockSpec(shape, lambda p, k: (0,) * len(shape))

    out_t = pl.pallas_call(
        _fused,
        grid=(2, NB),
        in_specs=[
            pl.BlockSpec((BLK, EMB), gate_blk),
            pl.BlockSpec((BLK, EMB), gate_blk),
            pl.BlockSpec((BLK, EMB), rep_blk),
            pl.BlockSpec((BLK, EMB), rep_blk),
            whole((GD, EMB)), whole((GD,)), whole((GD,)), whole((GD,)),
            whole((GD, GD)), whole((GD,)),
            whole((GD, EMB)), whole((GD,)), whole((GD,)), whole((GD,)),
            whole((GD, GD)), whole((GD,)),
            whole((E, GD)), whole((E, GD)),
            whole((AE, EMB)), whole((AE,)), whole((AE, EMB)), whole((AE,)),
        ],
        out_specs=pl.BlockSpec((T, BLK), lambda p, k: (0, k)),
        out_shape=jax.ShapeDtypeStruct((T, B), jnp.float32),
        scratch_shapes=[
            pltpu.VMEM((B, GD), jnp.float32),
            pltpu.VMEM((B, GD), jnp.float32),
            pltpu.VMEM((1, GD), jnp.float32),
            pltpu.VMEM((1, GD), jnp.float32),
            pltpu.VMEM((1, GD), jnp.float32),
            pltpu.VMEM((1, GD), jnp.float32),
        ],
        compiler_params=pltpu.CompilerParams(
            dimension_semantics=("arbitrary", "arbitrary"),
            vmem_limit_bytes=120 * 1024 * 1024),
    )(sca_gate_input, gro_gate_input, sca_rep, gro_rep,
      sca_g_w1.T, sca_g_b1, sca_g_gamma, sca_g_beta, sca_g_w2, sca_g_b2,
      gro_g_w1.T, gro_g_b1, gro_g_gamma, gro_g_beta, gro_g_w2, gro_g_b2,
      sca_cluster, gro_cluster,
      sca_experts_w.T, sca_experts_b, gro_experts_w.T, gro_experts_b)
    return out_t.T


# speedup vs baseline: 1.5168x; 1.0005x over previous
import jax
import jax.numpy as jnp
from jax.experimental import pallas as pl
from jax.experimental.pallas import tpu as pltpu

B = 4096
EMB = 128
GD = 300
E = 8
T = 12
AE = E * T
BLK = 512
NB = B // BLK


def _fused(xs_ref, xg_ref, rs_ref, rg_ref,
           sw1t_ref, sb1_ref, sgm_ref, sbt_ref, sw2_ref, sb2_ref,
           gw1t_ref, gb1_ref, ggm_ref, gbt_ref, gw2_ref, gb2_ref,
           sc_ref, gc_ref, sewt_ref, seb_ref, gewt_ref, geb_ref,
           out_ref,
           hs_scr, hg_scr, s1s_scr, s2s_scr, s1g_scr, s2g_scr,
           sw2b_scr, gw2b_scr):
    p = pl.program_id(0)
    k = pl.program_id(1)

    @pl.when(p == 0)
    def _phase0():
        def stats(x_ref, w1t_ref, b1_ref, h_scr, s1_scr, s2_scr):
            h = jax.lax.dot_general(
                x_ref[...], w1t_ref[...], (((1,), (1,)), ((), ())),
                preferred_element_type=jnp.float32) + b1_ref[...]
            h_scr[pl.ds(k * BLK, BLK), :] = h
            s1 = jnp.sum(h, axis=0, keepdims=True)
            s2 = jnp.sum(h * h, axis=0, keepdims=True)

            @pl.when(k == 0)
            def _init():
                s1_scr[...] = s1
                s2_scr[...] = s2

            @pl.when(k > 0)
            def _accum():
                s1_scr[...] += s1
                s2_scr[...] += s2

        stats(xs_ref, sw1t_ref, sb1_ref, hs_scr, s1s_scr, s2s_scr)
        stats(xg_ref, gw1t_ref, gb1_ref, hg_scr, s1g_scr, s2g_scr)

        @pl.when(k == 0)
        def _cvt_w2():
            sw2b_scr[...] = sw2_ref[...].astype(jnp.bfloat16)
            gw2b_scr[...] = gw2_ref[...].astype(jnp.bfloat16)

    @pl.when(p == 1)
    def _phase1():
        ie = jax.lax.broadcasted_iota(jnp.int32, (E, AE), 0)
        ja = jax.lax.broadcasted_iota(jnp.int32, (E, AE), 1)
        r_mat = (ja // T == ie).astype(jnp.float32)
        it = jax.lax.broadcasted_iota(jnp.int32, (T, AE), 0)
        jb = jax.lax.broadcasted_iota(jnp.int32, (T, AE), 1)
        st_mat = (jb % T == it).astype(jnp.float32)

        def half(rep, h_scr, s1_scr, s2_scr, gamma, beta, w2b, b2, clu,
                 ewt, eb):
            mu = s1_scr[...] * (1.0 / B)
            var = s2_scr[...] * (1.0 / B) - mu * mu
            scale = gamma * jax.lax.rsqrt(var + 1e-5)
            shift = beta - mu * scale
            h = h_scr[pl.ds(k * BLK, BLK), :] * scale + shift
            h = jnp.maximum(h, 0.0)
            ge = jnp.dot(h.astype(jnp.bfloat16), w2b,
                         preferred_element_type=jnp.float32) + b2
            logit = jax.lax.dot_general(
                ge, clu, (((1,), (1,)), ((), ())),
                preferred_element_type=jnp.float32)
            m = jnp.max(logit, axis=-1, keepdims=True)
            ex = jnp.exp(logit - m)
            q = ex / jnp.sum(ex, axis=-1, keepdims=True)
            z = jax.lax.dot_general(
                rep, ewt, (((1,), (1,)), ((), ())),
                preferred_element_type=jnp.float32) + eb
            qe = jnp.dot(q, r_mat, preferred_element_type=jnp.float32)
            return jax.lax.dot_general(
                st_mat, qe * z, (((1,), (1,)), ((), ())),
                preferred_element_type=jnp.float32)

        sca = half(rs_ref[...], hs_scr, s1s_scr, s2s_scr,
                   sgm_ref[...], sbt_ref[...], sw2b_scr[...], sb2_ref[...],
                   sc_ref[...], sewt_ref[...], seb_ref[...])
        gro = half(rg_ref[...], hg_scr, s1g_scr, s2g_scr,
                   ggm_ref[...], gbt_ref[...], gw2b_scr[...], gb2_ref[...],
                   gc_ref[...], gewt_ref[...], geb_ref[...])
        out_ref[...] = 0.5 * (sca + gro)


def kernel(sca_rep, gro_rep, sca_gate_input, gro_gate_input,
           sca_g_w1, sca_g_b1, sca_g_gamma, sca_g_beta, sca_g_w2, sca_g_b2,
           gro_g_w1, gro_g_b1, gro_g_gamma, gro_g_beta, gro_g_w2, gro_g_b2,
           sca_cluster, gro_cluster,
           sca_experts_w, sca_experts_b, gro_experts_w, gro_experts_b):
    gate_blk = lambda p, k: (jnp.where(p == 0, k, NB - 1), 0)
    rep_blk = lambda p, k: (jnp.where(p == 0, 0, k), 0)
    whole = lambda shape: pl.BlockSpec(shape, lambda p, k: (0,) * len(shape))

    out_t = pl.pallas_call(
        _fused,
        grid=(2, NB),
        in_specs=[
            pl.BlockSpec((BLK, EMB), gate_blk),
            pl.BlockSpec((BLK, EMB), gate_blk),
            pl.BlockSpec((BLK, EMB), rep_blk),
            pl.BlockSpec((BLK, EMB), rep_blk),
            whole((GD, EMB)), whole((GD,)), whole((GD,)), whole((GD,)),
            whole((GD, GD)), whole((GD,)),
            whole((GD, EMB)), whole((GD,)), whole((GD,)), whole((GD,)),
            whole((GD, GD)), whole((GD,)),
            whole((E, GD)), whole((E, GD)),
            whole((AE, EMB)), whole((AE,)), whole((AE, EMB)), whole((AE,)),
        ],
        out_specs=pl.BlockSpec((T, BLK), lambda p, k: (0, k)),
        out_shape=jax.ShapeDtypeStruct((T, B), jnp.float32),
        scratch_shapes=[
            pltpu.VMEM((B, GD), jnp.float32),
            pltpu.VMEM((B, GD), jnp.float32),
            pltpu.VMEM((1, GD), jnp.float32),
            pltpu.VMEM((1, GD), jnp.float32),
            pltpu.VMEM((1, GD), jnp.float32),
            pltpu.VMEM((1, GD), jnp.float32),
            pltpu.VMEM((GD, GD), jnp.bfloat16),
            pltpu.VMEM((GD, GD), jnp.bfloat16),
        ],
        compiler_params=pltpu.CompilerParams(
            dimension_semantics=("arbitrary", "arbitrary"),
            vmem_limit_bytes=120 * 1024 * 1024),
    )(sca_gate_input, gro_gate_input, sca_rep, gro_rep,
      sca_g_w1.T, sca_g_b1, sca_g_gamma, sca_g_beta, sca_g_w2, sca_g_b2,
      gro_g_w1.T, gro_g_b1, gro_g_gamma, gro_g_beta, gro_g_w2, gro_g_b2,
      sca_cluster, gro_cluster,
      sca_experts_w.T, sca_experts_b, gro_experts_w.T, gro_experts_b)
    return out_t.T


# speedup vs baseline: 1.9942x; 1.3147x over previous
import jax
import jax.numpy as jnp
from jax.experimental import pallas as pl
from jax.experimental.pallas import tpu as pltpu

B = 4096
EMB = 128
GD = 300
E = 8
T = 12
AE = E * T
CHUNK = 512
SB = 2 * CHUNK
NSTEP = B // SB


def _fused(xsa_ref, xsb_ref, xga_ref, xgb_ref,
           rsa_ref, rsb_ref, rga_ref, rgb_ref,
           sw1t_ref, sb1_ref, sgm_ref, sbt_ref, sw2_ref, sb2_ref,
           gw1t_ref, gb1_ref, ggm_ref, gbt_ref, gw2_ref, gb2_ref,
           sc_ref, gc_ref, sewt_ref, seb_ref, gewt_ref, geb_ref,
           out_ref,
           hs_scr, hg_scr, s1s_scr, s2s_scr, s1g_scr, s2g_scr,
           sw2b_scr, gw2b_scr):
    p = pl.program_id(0)
    k = pl.program_id(1)

    @pl.when(p == 0)
    def _phase0():
        def stats(xa_ref, xb_ref, w1t_ref, b1_ref, h_scr, s1_scr, s2_scr):
            x = jnp.concatenate([xa_ref[...], xb_ref[...]], axis=0)
            h = jax.lax.dot_general(
                x, w1t_ref[...], (((1,), (1,)), ((), ())),
                preferred_element_type=jnp.float32) + b1_ref[...]
            h_scr[pl.ds(k * SB, SB), :] = h
            s1 = jnp.sum(h, axis=0, keepdims=True)
            s2 = jnp.sum(h * h, axis=0, keepdims=True)

            @pl.when(k == 0)
            def _init():
                s1_scr[...] = s1
                s2_scr[...] = s2

            @pl.when(k > 0)
            def _accum():
                s1_scr[...] += s1
                s2_scr[...] += s2

        stats(xsa_ref, xsb_ref, sw1t_ref, sb1_ref, hs_scr, s1s_scr, s2s_scr)
        stats(xga_ref, xgb_ref, gw1t_ref, gb1_ref, hg_scr, s1g_scr, s2g_scr)

        @pl.when(k == 0)
        def _cvt_w2():
            sw2b_scr[...] = sw2_ref[...].astype(jnp.bfloat16)
            gw2b_scr[...] = gw2_ref[...].astype(jnp.bfloat16)

    @pl.when(p == 1)
    def _phase1():
        ie = jax.lax.broadcasted_iota(jnp.int32, (E, AE), 0)
        ja = jax.lax.broadcasted_iota(jnp.int32, (E, AE), 1)
        r_mat = (ja // T == ie).astype(jnp.float32)
        it = jax.lax.broadcasted_iota(jnp.int32, (T, AE), 0)
        jb = jax.lax.broadcasted_iota(jnp.int32, (T, AE), 1)
        st_mat = (jb % T == it).astype(jnp.float32)

        def half(ra_ref, rb_ref, h_scr, s1_scr, s2_scr, gamma, beta, w2b,
                 b2, clu, ewt, eb):
            mu = s1_scr[...] * (1.0 / B)
            var = s2_scr[...] * (1.0 / B) - mu * mu
            scale = gamma * jax.lax.rsqrt(var + 1e-5)
            shift = beta - mu * scale
            h = h_scr[pl.ds(k * SB, SB), :] * scale + shift
            h = jnp.maximum(h, 0.0)
            ge = jnp.dot(h.astype(jnp.bfloat16), w2b,
                         preferred_element_type=jnp.float32) + b2
            logit = jax.lax.dot_general(
                ge, clu, (((1,), (1,)), ((), ())),
                preferred_element_type=jnp.float32)
            m = jnp.max(logit, axis=-1, keepdims=True)
            ex = jnp.exp(logit - m)
            q = ex / jnp.sum(ex, axis=-1, keepdims=True)
            rep = jnp.concatenate([ra_ref[...], rb_ref[...]], axis=0)
            z = jax.lax.dot_general(
                rep, ewt, (((1,), (1,)), ((), ())),
                preferred_element_type=jnp.float32) + eb
            qe = jnp.dot(q, r_mat, preferred_element_type=jnp.float32)
            return jax.lax.dot_general(
                st_mat, qe * z, (((1,), (1,)), ((), ())),
                preferred_element_type=jnp.float32)

        sca = half(rsa_ref, rsb_ref, hs_scr, s1s_scr, s2s_scr,
                   sgm_ref[...], sbt_ref[...], sw2b_scr[...], sb2_ref[...],
                   sc_ref[...], sewt_ref[...], seb_ref[...])
        gro = half(rga_ref, rgb_ref, hg_scr, s1g_scr, s2g_scr,
                   ggm_ref[...], gbt_ref[...], gw2b_scr[...], gb2_ref[...],
                   gc_ref[...], gewt_ref[...], geb_ref[...])
        out_ref[...] = 0.5 * (sca + gro)


def kernel(sca_rep, gro_rep, sca_gate_input, gro_gate_input,
           sca_g_w1, sca_g_b1, sca_g_gamma, sca_g_beta, sca_g_w2, sca_g_b2,
           gro_g_w1, gro_g_b1, gro_g_gamma, gro_g_beta, gro_g_w2, gro_g_b2,
           sca_cluster, gro_cluster,
           sca_experts_w, sca_experts_b, gro_experts_w, gro_experts_b):
    gate_a = lambda p, k: (jnp.where(p == 0, 2 * k, 2 * NSTEP - 2), 0)
    gate_b = lambda p, k: (jnp.where(p == 0, 2 * k + 1, 2 * NSTEP - 1), 0)
    rep_a = lambda p, k: (jnp.where(p == 0, 0, 2 * k), 0)
    rep_b = lambda p, k: (jnp.where(p == 0, 1, 2 * k + 1), 0)
    whole = lambda shape: pl.BlockSpec(shape, lambda p, k: (0,) * len(shape))
    chunk = lambda im: pl.BlockSpec((CHUNK, EMB), im)

    out_t = pl.pallas_call(
        _fused,
        grid=(2, NSTEP),
        in_specs=[
            chunk(gate_a), chunk(gate_b),
            chunk(gate_a), chunk(gate_b),
            chunk(rep_a), chunk(rep_b),
            chunk(rep_a), chunk(rep_b),
            whole((GD, EMB)), whole((GD,)), whole((GD,)), whole((GD,)),
            whole((GD, GD)), whole((GD,)),
            whole((GD, EMB)), whole((GD,)), whole((GD,)), whole((GD,)),
            whole((GD, GD)), whole((GD,)),
            whole((E, GD)), whole((E, GD)),
            whole((AE, EMB)), whole((AE,)), whole((AE, EMB)), whole((AE,)),
        ],
        out_specs=pl.BlockSpec((T, SB), lambda p, k: (0, k)),
        out_shape=jax.ShapeDtypeStruct((T, B), jnp.float32),
        scratch_shapes=[
            pltpu.VMEM((B, GD), jnp.float32),
            pltpu.VMEM((B, GD), jnp.float32),
            pltpu.VMEM((1, GD), jnp.float32),
            pltpu.VMEM((1, GD), jnp.float32),
            pltpu.VMEM((1, GD), jnp.float32),
            pltpu.VMEM((1, GD), jnp.float32),
            pltpu.VMEM((GD, GD), jnp.bfloat16),
            pltpu.VMEM((GD, GD), jnp.bfloat16),
        ],
        compiler_params=pltpu.CompilerParams(
            dimension_semantics=("arbitrary", "arbitrary"),
            vmem_limit_bytes=120 * 1024 * 1024),
    )(sca_gate_input, sca_gate_input, gro_gate_input, gro_gate_input,
      sca_rep, sca_rep, gro_rep, gro_rep,
      sca_g_w1.T, sca_g_b1, sca_g_gamma, sca_g_beta, sca_g_w2, sca_g_b2,
      gro_g_w1.T, gro_g_b1, gro_g_gamma, gro_g_beta, gro_g_w2, gro_g_b2,
      sca_cluster, gro_cluster,
      sca_experts_w.T, sca_experts_b, gro_experts_w.T, gro_experts_b)
    return out_t.T


# speedup vs baseline: 2.0484x; 1.0272x over previous
import jax
import jax.numpy as jnp
from jax.experimental import pallas as pl
from jax.experimental.pallas import tpu as pltpu

B = 4096
EMB = 128
GD = 300
E = 8
T = 12
AE = E * T
CHUNK = 512
SB = 2 * CHUNK
NSTEP = B // SB


def _fused(xsa_ref, xsb_ref, xga_ref, xgb_ref,
           rsa_ref, rsb_ref, rga_ref, rgb_ref,
           sw1t_ref, sb1_ref, sgm_ref, sbt_ref, sw2_ref, sb2_ref,
           gw1t_ref, gb1_ref, ggm_ref, gbt_ref, gw2_ref, gb2_ref,
           sc_ref, gc_ref, sewt_ref, seb_ref, gewt_ref, geb_ref,
           out_ref,
           hs_scr, hg_scr, s1s_scr, s2s_scr, s1g_scr, s2g_scr,
           sw2b_scr, gw2b_scr):
    p = pl.program_id(0)
    k = pl.program_id(1)

    @pl.when(p == 0)
    def _phase0():
        def stats(xa_ref, xb_ref, w1t_ref, b1_ref, h_scr, s1_scr, s2_scr):
            x = jnp.concatenate([xa_ref[...], xb_ref[...]], axis=0)
            h = jax.lax.dot_general(
                x, w1t_ref[...], (((1,), (1,)), ((), ())),
                preferred_element_type=jnp.float32) + b1_ref[...]
            h_scr[pl.ds(k * SB, SB), :] = h.astype(jnp.bfloat16)
            s1 = jnp.sum(h, axis=0, keepdims=True)
            s2 = jnp.sum(h * h, axis=0, keepdims=True)

            @pl.when(k == 0)
            def _init():
                s1_scr[...] = s1
                s2_scr[...] = s2

            @pl.when(k > 0)
            def _accum():
                s1_scr[...] += s1
                s2_scr[...] += s2

        stats(xsa_ref, xsb_ref, sw1t_ref, sb1_ref, hs_scr, s1s_scr, s2s_scr)
        stats(xga_ref, xgb_ref, gw1t_ref, gb1_ref, hg_scr, s1g_scr, s2g_scr)

        @pl.when(k == 0)
        def _cvt_w2():
            sw2b_scr[...] = sw2_ref[...].astype(jnp.bfloat16)
            gw2b_scr[...] = gw2_ref[...].astype(jnp.bfloat16)

    @pl.when(p == 1)
    def _phase1():
        ie = jax.lax.broadcasted_iota(jnp.int32, (E, AE), 0)
        ja = jax.lax.broadcasted_iota(jnp.int32, (E, AE), 1)
        r_mat = (ja // T == ie).astype(jnp.float32)
        it = jax.lax.broadcasted_iota(jnp.int32, (T, AE), 0)
        jb = jax.lax.broadcasted_iota(jnp.int32, (T, AE), 1)
        st_mat = (jb % T == it).astype(jnp.float32)

        def half(ra_ref, rb_ref, h_scr, s1_scr, s2_scr, gamma, beta, w2b,
                 b2, clu, ewt, eb):
            mu = s1_scr[...] * (1.0 / B)
            var = s2_scr[...] * (1.0 / B) - mu * mu
            scale = (gamma * jax.lax.rsqrt(var + 1e-5)).astype(jnp.bfloat16)
            shift = (beta - mu * gamma * jax.lax.rsqrt(var + 1e-5)
                     ).astype(jnp.bfloat16)
            h = h_scr[pl.ds(k * SB, SB), :] * scale + shift
            h = jnp.maximum(h, jnp.bfloat16(0.0))
            ge = jnp.dot(h, w2b, preferred_element_type=jnp.float32) + b2
            logit_t = jax.lax.dot_general(
                clu, ge, (((1,), (1,)), ((), ())),
                preferred_element_type=jnp.float32)
            m = jnp.max(logit_t, axis=0, keepdims=True)
            ex = jnp.exp(logit_t - m)
            q_t = ex / jnp.sum(ex, axis=0, keepdims=True)
            rep = jnp.concatenate([ra_ref[...], rb_ref[...]], axis=0)
            z = jax.lax.dot_general(
                rep, ewt, (((1,), (1,)), ((), ())),
                preferred_element_type=jnp.float32) + eb
            qe = jax.lax.dot_general(
                q_t, r_mat, (((0,), (0,)), ((), ())),
                preferred_element_type=jnp.float32)
            return jax.lax.dot_general(
                st_mat, qe * z, (((1,), (1,)), ((), ())),
                preferred_element_type=jnp.float32)

        sca = half(rsa_ref, rsb_ref, hs_scr, s1s_scr, s2s_scr,
                   sgm_ref[...], sbt_ref[...], sw2b_scr[...], sb2_ref[...],
                   sc_ref[...], sewt_ref[...], seb_ref[...])
        gro = half(rga_ref, rgb_ref, hg_scr, s1g_scr, s2g_scr,
                   ggm_ref[...], gbt_ref[...], gw2b_scr[...], gb2_ref[...],
                   gc_ref[...], gewt_ref[...], geb_ref[...])
        out_ref[...] = 0.5 * (sca + gro)


def kernel(sca_rep, gro_rep, sca_gate_input, gro_gate_input,
           sca_g_w1, sca_g_b1, sca_g_gamma, sca_g_beta, sca_g_w2, sca_g_b2,
           gro_g_w1, gro_g_b1, gro_g_gamma, gro_g_beta, gro_g_w2, gro_g_b2,
           sca_cluster, gro_cluster,
           sca_experts_w, sca_experts_b, gro_experts_w, gro_experts_b):
    gate_a = lambda p, k: (jnp.where(p == 0, 2 * k, 2 * NSTEP - 2), 0)
    gate_b = lambda p, k: (jnp.where(p == 0, 2 * k + 1, 2 * NSTEP - 1), 0)
    rep_a = lambda p, k: (jnp.where(p == 0, 0, 2 * k), 0)
    rep_b = lambda p, k: (jnp.where(p == 0, 1, 2 * k + 1), 0)
    whole = lambda shape: pl.BlockSpec(shape, lambda p, k: (0,) * len(shape))
    chunk = lambda im: pl.BlockSpec((CHUNK, EMB), im)

    out_t = pl.pallas_call(
        _fused,
        grid=(2, NSTEP),
        in_specs=[
            chunk(gate_a), chunk(gate_b),
            chunk(gate_a), chunk(gate_b),
            chunk(rep_a), chunk(rep_b),
            chunk(rep_a), chunk(rep_b),
            whole((GD, EMB)), whole((GD,)), whole((GD,)), whole((GD,)),
            whole((GD, GD)), whole((GD,)),
            whole((GD, EMB)), whole((GD,)), whole((GD,)), whole((GD,)),
            whole((GD, GD)), whole((GD,)),
            whole((E, GD)), whole((E, GD)),
            whole((AE, EMB)), whole((AE,)), whole((AE, EMB)), whole((AE,)),
        ],
        out_specs=pl.BlockSpec((T, SB), lambda p, k: (0, k)),
        out_shape=jax.ShapeDtypeStruct((T, B), jnp.float32),
        scratch_shapes=[
            pltpu.VMEM((B, GD), jnp.bfloat16),
            pltpu.VMEM((B, GD), jnp.bfloat16),
            pltpu.VMEM((1, GD), jnp.float32),
            pltpu.VMEM((1, GD), jnp.float32),
            pltpu.VMEM((1, GD), jnp.float32),
            pltpu.VMEM((1, GD), jnp.float32),
            pltpu.VMEM((GD, GD), jnp.bfloat16),
            pltpu.VMEM((GD, GD), jnp.bfloat16),
        ],
        compiler_params=pltpu.CompilerParams(
            dimension_semantics=("arbitrary", "arbitrary"),
            vmem_limit_bytes=120 * 1024 * 1024),
    )(sca_gate_input, sca_gate_input, gro_gate_input, gro_gate_input,
      sca_rep, sca_rep, gro_rep, gro_rep,
      sca_g_w1.T, sca_g_b1, sca_g_gamma, sca_g_beta, sca_g_w2, sca_g_b2,
      gro_g_w1.T, gro_g_b1, gro_g_gamma, gro_g_beta, gro_g_w2, gro_g_b2,
      sca_cluster, gro_cluster,
      sca_experts_w.T, sca_experts_b, gro_experts_w.T, gro_experts_b)
    return out_t.T
